# baseline (speedup 1.0000x reference)
import functools

import jax
import jax.numpy as jnp
from jax import lax
from jax.experimental import pallas as pl
from jax.experimental.pallas import tpu as pltpu
from jax.experimental.pallas import tpu_sc as plsc

N_USERS = 4000
N_ITEMS = 6000
N_NODES = N_USERS + N_ITEMS
DIM = 128
E = 320000
N_LAYERS = 3
B = 4096

NC = 2
NS = 16
NW = NC * NS
EW = E // NW
C = 128
EWP = 10240
NCH = EWP // C
NPAD = 10240
RW = NPAD // NS
RCH = 128
NZ = RW // RCH
NLANE = DIM // 16
BW = B // NW

_mesh = plsc.VectorSubcoreMesh(core_axis_name="c", subcore_axis_name="s",
                               num_cores=NC, num_subcores=NS)


def _build_prop(interpret=False):
    @functools.partial(
        pl.kernel,
        out_type=jax.ShapeDtypeStruct((NC, NPAD, DIM), jnp.float32),
        mesh=_mesh,
        interpret=interpret,
        scratch_types=[
            pltpu.VMEM((NCH, C), jnp.int32),
            pltpu.VMEM((2 * C,), jnp.int32),
            pltpu.VMEM((2 * C,), jnp.float32),
            pltpu.VMEM((2, C, DIM), jnp.float32),
            pltpu.VMEM_SHARED((NPAD, DIM), jnp.float32),
            pltpu.SemaphoreType.DMA,
            pltpu.SemaphoreType.DMA,
            pltpu.SemaphoreType.DMA,
            pltpu.SemaphoreType.DMA,
            pltpu.SemaphoreType.DMA,
            pltpu.SemaphoreType.DMA,
        ],
    )
    def prop(emb_hbm, src_hbm, dst_hbm, w_hbm, partial_hbm,
             dst_v, sb_v, wb_v, rows_v, acc,
             csem0, csem1, gsem0, gsem1, ssem0, ssem1):
        csem = (csem0, csem1)
        gsem = (gsem0, gsem1)
        ssem = (ssem0, ssem1)
        c = lax.axis_index("c")
        s = lax.axis_index("s")
        pltpu.sync_copy(dst_hbm.at[c, s], dst_v)

        zeros16 = jnp.zeros((16,), jnp.float32)

        @pl.loop(0, C)
        def _zero(r):
            for d in range(NLANE):
                rows_v[0, r, pl.ds(d * 16, 16)] = zeros16

        for k in range(NZ):
            pltpu.sync_copy(rows_v.at[0], acc.at[pl.ds(s * RW + k * C, C)])
        plsc.subcore_barrier()

        def fetch_idx(jj, b):
            pltpu.async_copy(src_hbm.at[c, s, jj], sb_v.at[pl.ds(b * C, C)],
                             csem[b])
            pltpu.async_copy(w_hbm.at[c, s, jj], wb_v.at[pl.ds(b * C, C)],
                             csem[b])

        def wait_idx(jj, b):
            pltpu.make_async_copy(src_hbm.at[c, s, jj],
                                  sb_v.at[pl.ds(b * C, C)], csem[b]).wait()
            pltpu.make_async_copy(w_hbm.at[c, s, jj],
                                  wb_v.at[pl.ds(b * C, C)], csem[b]).wait()

        H = C // 2

        def start_gather(jj, b):
            pltpu.async_copy(emb_hbm.at[sb_v.at[pl.ds(b * C, H)]],
                             rows_v.at[b, pl.ds(0, H)], gsem[b])
            pltpu.async_copy(emb_hbm.at[sb_v.at[pl.ds(b * C + H, H)]],
                             rows_v.at[b, pl.ds(H, H)], gsem[b])

        def wait_gather(jj, b):
            pltpu.make_async_copy(emb_hbm.at[sb_v.at[pl.ds(b * C, H)]],
                                  rows_v.at[b, pl.ds(0, H)], gsem[b]).wait()
            pltpu.make_async_copy(emb_hbm.at[sb_v.at[pl.ds(b * C + H, H)]],
                                  rows_v.at[b, pl.ds(H, H)], gsem[b]).wait()

        def start_scatter(jj, b):
            pltpu.async_copy(rows_v.at[b], acc.at[dst_v.at[jj]], ssem[b],
                             add=True)

        def wait_scatter(jj, b):
            pltpu.make_async_copy(rows_v.at[b], acc.at[dst_v.at[jj]],
                                  ssem[b]).wait()

        fetch_idx(0, 0)
        fetch_idx(1, 1)
        wait_idx(0, 0)
        start_gather(0, 0)
        wait_idx(1, 1)
        start_gather(1, 1)

        @pl.loop(0, NCH, step=2)
        def _outer(j):
            for b in range(2):
                jj = j + b
                wait_gather(jj, b)

                @pl.loop(0, C // 16)
                def _scale(g):
                    wvec = wb_v[pl.ds(b * C + g * 16, 16)]
                    for e16 in range(16):
                        wv = wvec[e16]
                        e = g * 16 + e16
                        for d in range(NLANE):
                            sl = pl.ds(d * 16, 16)
                            rows_v[b, e, sl] = rows_v[b, e, sl] * wv

                start_scatter(jj, b)

                @pl.when(jj + 2 < NCH)
                def _pf():
                    fetch_idx(jj + 2, b)

                @pl.when(jj + 2 < NCH)
                def _nxt():
                    wait_idx(jj + 2, b)
                    wait_scatter(jj, b)
                    start_gather(jj + 2, b)

                @pl.when(jj + 2 >= NCH)
                def _drain():
                    wait_scatter(jj, b)

        plsc.subcore_barrier()
        pltpu.sync_copy(acc.at[pl.ds(s * RW, RW)],
                        partial_hbm.at[c, pl.ds(s * RW, RW)])

    return prop


def _build_score(interpret=False):
    @functools.partial(
        pl.kernel,
        out_type=(jax.ShapeDtypeStruct((B, DIM), jnp.float32),
                  jax.ShapeDtypeStruct((B, DIM), jnp.float32)),
        mesh=_mesh,
        interpret=interpret,
        scratch_types=[
            pltpu.VMEM((BW,), jnp.int32),
            pltpu.VMEM((BW,), jnp.int32),
            pltpu.VMEM((BW,), jnp.int32),
            pltpu.VMEM((BW, DIM), jnp.float32),
            pltpu.VMEM((BW, DIM), jnp.float32),
            pltpu.VMEM((BW, DIM), jnp.float32),
            pltpu.SemaphoreType.DMA,
        ],
    )
    def score(sum_hbm, u_hbm, i_hbm, n_hbm, pp_hbm, np_hbm,
              ui_v, ii_v, ni_v, ur_v, ir_v, nr_v, sem):
        c = lax.axis_index("c")
        s = lax.axis_index("s")
        wid = s * NC + c
        base = wid * BW
        pltpu.sync_copy(u_hbm.at[pl.ds(base, BW)], ui_v)
        pltpu.sync_copy(i_hbm.at[pl.ds(base, BW)], ii_v)
        pltpu.sync_copy(n_hbm.at[pl.ds(base, BW)], ni_v)
        pltpu.async_copy(sum_hbm.at[ui_v], ur_v, sem).wait()
        pltpu.async_copy(sum_hbm.at[ii_v], ir_v, sem).wait()
        pltpu.async_copy(sum_hbm.at[ni_v], nr_v, sem).wait()

        @pl.loop(0, BW)
        def _mul(e):
            for d in range(NLANE):
                sl = pl.ds(d * 16, 16)
                uv = ur_v[e, sl]
                ir_v[e, sl] = uv * ir_v[e, sl]
                nr_v[e, sl] = uv * nr_v[e, sl]

        pltpu.sync_copy(ir_v, pp_hbm.at[pl.ds(base, BW)])
        pltpu.sync_copy(nr_v, np_hbm.at[pl.ds(base, BW)])

    return score


def _dot_tc_body(pp_ref, np_ref, pos_ref, neg_ref):
    pos_ref[...] = jnp.sum(pp_ref[...], axis=2) * 0.0625
    neg_ref[...] = jnp.sum(np_ref[...], axis=2) * 0.0625


_dot_tc = pl.pallas_call(
    _dot_tc_body,
    out_shape=(jax.ShapeDtypeStruct((B // DIM, DIM), jnp.float32),
               jax.ShapeDtypeStruct((B // DIM, DIM), jnp.float32)),
)


def _comb_tc_body(p_ref, s_ref, e_ref, o_ref):
    a = p_ref[0] + p_ref[1]
    e_ref[...] = a
    o_ref[...] = s_ref[...] + a


_combine_tc = pl.pallas_call(
    _comb_tc_body,
    grid=(NPAD // 1024,),
    in_specs=[pl.BlockSpec((2, 1024, DIM), lambda i: (0, i, 0)),
              pl.BlockSpec((1024, DIM), lambda i: (i, 0))],
    out_specs=[pl.BlockSpec((1024, DIM), lambda i: (i, 0)),
               pl.BlockSpec((1024, DIM), lambda i: (i, 0))],
    out_shape=(jax.ShapeDtypeStruct((NPAD, DIM), jnp.float32),
               jax.ShapeDtypeStruct((NPAD, DIM), jnp.float32)),
)


_prop = _build_prop()
_score = _build_score()


def kernel(user_emb, item_emb, edge_weight, edge_index, u, i, neg_i):
    npad_e = EWP - EW
    src = jnp.concatenate(
        [edge_index[0].astype(jnp.int32).reshape(NW, EW),
         jnp.zeros((NW, npad_e), jnp.int32)], axis=1
    ).reshape(NC, NS, NCH, C)
    dst = jnp.concatenate(
        [edge_index[1].astype(jnp.int32).reshape(NW, EW),
         jnp.full((NW, npad_e), N_NODES, jnp.int32)], axis=1
    ).reshape(NC, NS, NCH, C)
    w = jnp.concatenate(
        [edge_weight.astype(jnp.float32).reshape(NW, EW),
         jnp.zeros((NW, npad_e), jnp.float32)], axis=1
    ).reshape(NC, NS, NCH, C)
    emb = jnp.concatenate(
        [user_emb, item_emb,
         jnp.zeros((NPAD - N_NODES, DIM), jnp.float32)], axis=0
    ).astype(jnp.float32)
    u_idx = u.astype(jnp.int32)
    i_idx = i.astype(jnp.int32) + N_USERS
    n_idx = neg_i.astype(jnp.int32) + N_USERS

    sum_emb = emb
    for _ in range(N_LAYERS):
        partial = _prop(emb, src, dst, w)
        emb, sum_emb = _combine_tc(partial, sum_emb)
    pp, npr = _score(sum_emb, u_idx, i_idx, n_idx)
    pos, neg = _dot_tc(pp.reshape(B // DIM, DIM, DIM),
                       npr.reshape(B // DIM, DIM, DIM))
    return (pos.reshape(B), neg.reshape(B))


# speedup vs baseline: 1.0010x; 1.0010x over previous
import functools

import jax
import jax.numpy as jnp
from jax import lax
from jax.experimental import pallas as pl
from jax.experimental.pallas import tpu as pltpu
from jax.experimental.pallas import tpu_sc as plsc

N_USERS = 4000
N_ITEMS = 6000
N_NODES = N_USERS + N_ITEMS
DIM = 128
E = 320000
N_LAYERS = 3
B = 4096

NC = 2
NS = 16
NW = NC * NS
EW = E // NW
C = 128
EWP = 10240
NCH = EWP // C
NPAD = 10240
RW = NPAD // NS
RCH = 128
NZ = RW // RCH
NLANE = DIM // 16
BW = B // NW

_mesh = plsc.VectorSubcoreMesh(core_axis_name="c", subcore_axis_name="s",
                               num_cores=NC, num_subcores=NS)


def _build_prop(interpret=False):
    @functools.partial(
        pl.kernel,
        out_type=jax.ShapeDtypeStruct((NC, NPAD, DIM), jnp.float32),
        mesh=_mesh,
        interpret=interpret,
        scratch_types=[
            pltpu.VMEM((NCH, C), jnp.int32),
            pltpu.VMEM((2 * C,), jnp.int32),
            pltpu.VMEM((2 * C,), jnp.float32),
            pltpu.VMEM((2, C, DIM), jnp.float32),
            pltpu.VMEM_SHARED((NPAD, DIM), jnp.float32),
            pltpu.SemaphoreType.DMA,
            pltpu.SemaphoreType.DMA,
            pltpu.SemaphoreType.DMA,
            pltpu.SemaphoreType.DMA,
            pltpu.SemaphoreType.DMA,
            pltpu.SemaphoreType.DMA,
        ],
    )
    def prop(emb_hbm, src_hbm, dst_hbm, w_hbm, partial_hbm,
             dst_v, sb_v, wb_v, rows_v, acc,
             csem0, csem1, gsem0, gsem1, ssem0, ssem1):
        csem = (csem0, csem1)
        gsem = (gsem0, gsem1)
        ssem = (ssem0, ssem1)
        c = lax.axis_index("c")
        s = lax.axis_index("s")
        pltpu.sync_copy(dst_hbm.at[c, s], dst_v)

        zeros16 = jnp.zeros((16,), jnp.float32)

        @pl.loop(0, C)
        def _zero(r):
            for d in range(NLANE):
                rows_v[0, r, pl.ds(d * 16, 16)] = zeros16

        for k in range(NZ):
            pltpu.sync_copy(rows_v.at[0], acc.at[pl.ds(s * RW + k * C, C)])
        plsc.subcore_barrier()

        def fetch_src(jj, b):
            pltpu.async_copy(src_hbm.at[c, s, jj], sb_v.at[pl.ds(b * C, C)],
                             csem[b])

        def fetch_w(jj, b):
            pltpu.async_copy(w_hbm.at[c, s, jj], wb_v.at[pl.ds(b * C, C)],
                             csem[b])

        def fetch_idx(jj, b):
            fetch_src(jj, b)
            fetch_w(jj, b)

        def wait_idx(jj, b):
            pltpu.make_async_copy(src_hbm.at[c, s, jj],
                                  sb_v.at[pl.ds(b * C, C)], csem[b]).wait()
            pltpu.make_async_copy(w_hbm.at[c, s, jj],
                                  wb_v.at[pl.ds(b * C, C)], csem[b]).wait()

        H = C // 2

        def start_gather(jj, b):
            pltpu.async_copy(emb_hbm.at[sb_v.at[pl.ds(b * C, H)]],
                             rows_v.at[b, pl.ds(0, H)], gsem[b])
            pltpu.async_copy(emb_hbm.at[sb_v.at[pl.ds(b * C + H, H)]],
                             rows_v.at[b, pl.ds(H, H)], gsem[b])

        def wait_gather(jj, b):
            pltpu.make_async_copy(emb_hbm.at[sb_v.at[pl.ds(b * C, H)]],
                                  rows_v.at[b, pl.ds(0, H)], gsem[b]).wait()
            pltpu.make_async_copy(emb_hbm.at[sb_v.at[pl.ds(b * C + H, H)]],
                                  rows_v.at[b, pl.ds(H, H)], gsem[b]).wait()

        def start_scatter(jj, b):
            pltpu.async_copy(rows_v.at[b], acc.at[dst_v.at[jj]], ssem[b],
                             add=True)

        def wait_scatter(jj, b):
            pltpu.make_async_copy(rows_v.at[b], acc.at[dst_v.at[jj]],
                                  ssem[b]).wait()

        fetch_idx(0, 0)
        fetch_idx(1, 1)
        wait_idx(0, 0)
        start_gather(0, 0)
        wait_idx(1, 1)
        start_gather(1, 1)

        @pl.loop(0, NCH, step=2)
        def _outer(j):
            for b in range(2):
                jj = j + b
                wait_gather(jj, b)

                @pl.when(jj + 2 < NCH)
                def _pfs():
                    fetch_src(jj + 2, b)

                @pl.loop(0, C // 16)
                def _scale(g):
                    wvec = wb_v[pl.ds(b * C + g * 16, 16)]
                    for e16 in range(16):
                        wv = wvec[e16]
                        e = g * 16 + e16
                        for d in range(NLANE):
                            sl = pl.ds(d * 16, 16)
                            rows_v[b, e, sl] = rows_v[b, e, sl] * wv

                start_scatter(jj, b)

                @pl.when(jj + 2 < NCH)
                def _pfw():
                    fetch_w(jj + 2, b)

                @pl.when(jj + 2 < NCH)
                def _nxt():
                    wait_idx(jj + 2, b)
                    wait_scatter(jj, b)
                    start_gather(jj + 2, b)

                @pl.when(jj + 2 >= NCH)
                def _drain():
                    wait_scatter(jj, b)

        plsc.subcore_barrier()
        pltpu.sync_copy(acc.at[pl.ds(s * RW, RW)],
                        partial_hbm.at[c, pl.ds(s * RW, RW)])

    return prop


def _build_score(interpret=False):
    @functools.partial(
        pl.kernel,
        out_type=(jax.ShapeDtypeStruct((B, DIM), jnp.float32),
                  jax.ShapeDtypeStruct((B, DIM), jnp.float32)),
        mesh=_mesh,
        interpret=interpret,
        scratch_types=[
            pltpu.VMEM((BW,), jnp.int32),
            pltpu.VMEM((BW,), jnp.int32),
            pltpu.VMEM((BW,), jnp.int32),
            pltpu.VMEM((BW, DIM), jnp.float32),
            pltpu.VMEM((BW, DIM), jnp.float32),
            pltpu.VMEM((BW, DIM), jnp.float32),
            pltpu.SemaphoreType.DMA,
        ],
    )
    def score(sum_hbm, u_hbm, i_hbm, n_hbm, pp_hbm, np_hbm,
              ui_v, ii_v, ni_v, ur_v, ir_v, nr_v, sem):
        c = lax.axis_index("c")
        s = lax.axis_index("s")
        wid = s * NC + c
        base = wid * BW
        pltpu.sync_copy(u_hbm.at[pl.ds(base, BW)], ui_v)
        pltpu.sync_copy(i_hbm.at[pl.ds(base, BW)], ii_v)
        pltpu.sync_copy(n_hbm.at[pl.ds(base, BW)], ni_v)
        pltpu.async_copy(sum_hbm.at[ui_v], ur_v, sem).wait()
        pltpu.async_copy(sum_hbm.at[ii_v], ir_v, sem).wait()
        pltpu.async_copy(sum_hbm.at[ni_v], nr_v, sem).wait()

        @pl.loop(0, BW)
        def _mul(e):
            for d in range(NLANE):
                sl = pl.ds(d * 16, 16)
                uv = ur_v[e, sl]
                ir_v[e, sl] = uv * ir_v[e, sl]
                nr_v[e, sl] = uv * nr_v[e, sl]

        pltpu.sync_copy(ir_v, pp_hbm.at[pl.ds(base, BW)])
        pltpu.sync_copy(nr_v, np_hbm.at[pl.ds(base, BW)])

    return score


def _dot_tc_body(pp_ref, np_ref, pos_ref, neg_ref):
    pos_ref[...] = jnp.sum(pp_ref[...], axis=2) * 0.0625
    neg_ref[...] = jnp.sum(np_ref[...], axis=2) * 0.0625


_dot_tc = pl.pallas_call(
    _dot_tc_body,
    out_shape=(jax.ShapeDtypeStruct((B // DIM, DIM), jnp.float32),
               jax.ShapeDtypeStruct((B // DIM, DIM), jnp.float32)),
)


def _comb_tc_body(p_ref, s_ref, e_ref, o_ref):
    a = p_ref[0] + p_ref[1]
    e_ref[...] = a
    o_ref[...] = s_ref[...] + a


_combine_tc = pl.pallas_call(
    _comb_tc_body,
    grid=(NPAD // 1024,),
    in_specs=[pl.BlockSpec((2, 1024, DIM), lambda i: (0, i, 0)),
              pl.BlockSpec((1024, DIM), lambda i: (i, 0))],
    out_specs=[pl.BlockSpec((1024, DIM), lambda i: (i, 0)),
               pl.BlockSpec((1024, DIM), lambda i: (i, 0))],
    out_shape=(jax.ShapeDtypeStruct((NPAD, DIM), jnp.float32),
               jax.ShapeDtypeStruct((NPAD, DIM), jnp.float32)),
)


_prop = _build_prop()
_score = _build_score()


def kernel(user_emb, item_emb, edge_weight, edge_index, u, i, neg_i):
    npad_e = EWP - EW
    src = jnp.concatenate(
        [edge_index[0].astype(jnp.int32).reshape(NW, EW),
         jnp.zeros((NW, npad_e), jnp.int32)], axis=1
    ).reshape(NC, NS, NCH, C)
    dst = jnp.concatenate(
        [edge_index[1].astype(jnp.int32).reshape(NW, EW),
         jnp.full((NW, npad_e), N_NODES, jnp.int32)], axis=1
    ).reshape(NC, NS, NCH, C)
    w = jnp.concatenate(
        [edge_weight.astype(jnp.float32).reshape(NW, EW),
         jnp.zeros((NW, npad_e), jnp.float32)], axis=1
    ).reshape(NC, NS, NCH, C)
    emb = jnp.concatenate(
        [user_emb, item_emb,
         jnp.zeros((NPAD - N_NODES, DIM), jnp.float32)], axis=0
    ).astype(jnp.float32)
    u_idx = u.astype(jnp.int32)
    i_idx = i.astype(jnp.int32) + N_USERS
    n_idx = neg_i.astype(jnp.int32) + N_USERS

    sum_emb = emb
    for _ in range(N_LAYERS):
        partial = _prop(emb, src, dst, w)
        emb, sum_emb = _combine_tc(partial, sum_emb)
    pp, npr = _score(sum_emb, u_idx, i_idx, n_idx)
    pos, neg = _dot_tc(pp.reshape(B // DIM, DIM, DIM),
                       npr.reshape(B // DIM, DIM, DIM))
    return (pos.reshape(B), neg.reshape(B))
